# Initial kernel scaffold; baseline (speedup 1.0000x reference)
import jax
import jax.numpy as jnp
from jax.experimental import pallas as pl


def kernel(x1, x2, edge1_1, edge1_2, edge2_1, edge2_2, index1_1, index1_2, index2_1, index2_2, Wc11, bc11, Wc12, bc12, Wc21, bc21, Wc22, bc22, W_m1a, b_m1a, W_m1b, b_m1b, W_m2a, b_m2a, W_m2b, b_m2b, W_ma, b_ma, W_mb, b_mb, Wd1, bd1, Wd2, bd2, Wd3, bd3):
    raise NotImplementedError("write your pallas kernel here")


# speedup vs baseline: 5.9927x; 5.9927x over previous
import functools

import jax
import jax.numpy as jnp
from jax import lax
from jax.experimental import pallas as pl
from jax.experimental.pallas import tpu as pltpu
from jax.experimental.pallas import tpu_sc as plsc

N = 10000
E = 160000
D = 256
H = 256
S = 64

NP = 10240
NT = 16
NC = 2
ROWS_T = NP // NT
CHUNK = 128
EPT = 10112
NITER = EPT // CHUNK
EPAD = EPT * NT
PAD = EPAD - E
HB = 64
EPT_H = (EPAD // NC) // NT
NITER_H = EPT_H // HB


def _deg_body(d1, d2, d3, d4, ones_h, z1, out, idxb, ones_v, acc):
    c = lax.axis_index("c")
    s = lax.axis_index("s")
    pltpu.sync_copy(ones_h, ones_v)
    zoff = pl.multiple_of(s * ROWS_T, 8)
    tbase = c * (EPAD // NC) + s * EPT_H
    for k, dk in enumerate((d1, d2, d3, d4)):
        pltpu.sync_copy(z1, acc.at[pl.ds(zoff, ROWS_T)])
        plsc.subcore_barrier()

        def body(i, _, dk=dk):
            base = pl.multiple_of(tbase + i * HB, 8)
            pltpu.sync_copy(dk.at[pl.ds(base, HB)], idxb)
            pltpu.sync_copy(ones_v, acc.at[idxb], add=True)
            return 0

        lax.fori_loop(0, NITER_H, body, 0)
        plsc.subcore_barrier()
        ooff = pl.multiple_of((c * 4 + k) * NP + s * ROWS_T, 8)
        pltpu.sync_copy(acc.at[pl.ds(zoff, ROWS_T)], out.at[pl.ds(ooff, ROWS_T)])
        plsc.subcore_barrier()


@functools.cache
def _sc_kernels():
    mesh = plsc.VectorSubcoreMesh(core_axis_name="c", subcore_axis_name="s",
                                  num_cores=NC, num_subcores=NT)
    deg = pl.kernel(
        _deg_body,
        out_type=jax.ShapeDtypeStruct((NC * 4 * NP, 8), jnp.float32),
        mesh=mesh,
        scratch_types=[
            pltpu.VMEM((HB,), jnp.int32),
            pltpu.VMEM((HB, 8), jnp.float32),
            pltpu.VMEM_SHARED((NP, 8), jnp.float32),
        ],
    )
    scat = pl.kernel(
        _scat_body,
        out_type=jax.ShapeDtypeStruct((4 * NP, 128), jnp.float32),
        mesh=mesh,
        scratch_types=[
            pltpu.VMEM((CHUNK,), jnp.int32),
            pltpu.VMEM((CHUNK,), jnp.int32),
            pltpu.VMEM((CHUNK, 128), jnp.float32),
            pltpu.VMEM_SHARED((NP, 128), jnp.float32),
            pltpu.SemaphoreType.DMA,
        ],
    )
    return deg, scat


def _scat_body(hpre, sx1, d1, sx2, d2, z, out,
               idx_g, idx_d, rows, accum, sem):
    c = lax.axis_index("c")
    s = lax.axis_index("s")
    zoff = pl.multiple_of(s * ROWS_T, 8)
    for k, (sxk, dk) in enumerate(((sx1, d1), (sx2, d2))):
        pltpu.sync_copy(z, accum.at[pl.ds(zoff, ROWS_T)])
        plsc.subcore_barrier()

        def body(i, _, sxk=sxk, dk=dk):
            sbase = pl.multiple_of(c * EPAD + s * EPT + i * CHUNK, 8)
            dbase = pl.multiple_of(s * EPT + i * CHUNK, 8)
            pltpu.sync_copy(sxk.at[pl.ds(sbase, CHUNK)], idx_g)
            pltpu.sync_copy(dk.at[pl.ds(dbase, CHUNK)], idx_d)
            pltpu.async_copy(hpre.at[idx_g], rows, sem).wait()
            pltpu.sync_copy(rows, accum.at[idx_d], add=True)
            return 0

        lax.fori_loop(0, NITER, body, 0)
        plsc.subcore_barrier()
        ooff = pl.multiple_of((2 * k + c) * NP + s * ROWS_T, 8)
        pltpu.sync_copy(accum.at[pl.ds(zoff, ROWS_T)],
                        out.at[pl.ds(ooff, ROWS_T)])
        plsc.subcore_barrier()


def _k1_body(x_ref, w_ref, cnt_ref, hg_ref, hp_ref):
    deg = 1.0 + cnt_ref[0, :, 0] + cnt_ref[0, :, 1]
    dinv = lax.rsqrt(deg)
    h = jnp.dot(x_ref[...], w_ref[0], preferred_element_type=jnp.float32)
    hg_ref[0, 0] = h
    hp_ref[0, 0] = h * dinv[:, None]


def _k1(x, w2, cnt):
    bs = 1000
    grid = (2, 2, N // bs)
    return pl.pallas_call(
        _k1_body,
        grid=grid,
        in_specs=[
            pl.BlockSpec((bs, D), lambda e, c, i: (i, 0)),
            pl.BlockSpec((1, D, 128), lambda e, c, i: (e, 0, c)),
            pl.BlockSpec((1, bs, 2), lambda e, c, i: (e, i, 0)),
        ],
        out_specs=[
            pl.BlockSpec((1, 1, bs, 128), lambda e, c, i: (e, c, i, 0)),
            pl.BlockSpec((1, 1, bs, 128), lambda e, c, i: (e, c, i, 0)),
        ],
        out_shape=[
            jax.ShapeDtypeStruct((2, 2, N, 128), jnp.float32),
            jax.ShapeDtypeStruct((2, 2, N, 128), jnp.float32),
        ],
    )(x, w2, cnt)


def _gcn_cat(scat, hg, cnt_ref, bc_ref):
    outs = []
    for e in range(2):
        deg = 1.0 + cnt_ref[e, :, 0] + cnt_ref[e, :, 1]
        dinv = lax.rsqrt(deg)[:, None]
        sc = jnp.concatenate([scat[e, 0], scat[e, 1]], axis=1)
        hgc = jnp.concatenate([hg[e, 0], hg[e, 1]], axis=1)
        g = dinv * sc + (dinv * dinv) * hgc + bc_ref[e][None, :]
        outs.append(jnp.maximum(g, 0.0))
    return jnp.concatenate(outs, axis=1)


def _k2_body(scat_ref, hg_ref, cnt_ref, bc_ref, wa_ref, ba_ref, wb_ref,
             bb_ref, wc2_ref, hg2_ref, hp2_ref):
    cat = _gcn_cat(scat_ref[...], hg_ref[...], cnt_ref, bc_ref)
    u = jnp.maximum(
        jnp.dot(cat, wa_ref[...], preferred_element_type=jnp.float32)
        + ba_ref[0][None, :], 0.0)
    t = jnp.dot(u, wb_ref[...], preferred_element_type=jnp.float32) \
        + bb_ref[0][None, :]
    for e in range(2):
        deg = 1.0 + cnt_ref[e, :, 0] + cnt_ref[e, :, 1]
        dinv = lax.rsqrt(deg)[:, None]
        h2 = jnp.dot(t, wc2_ref[e], preferred_element_type=jnp.float32)
        hg2_ref[e, 0] = h2[:, :128]
        hg2_ref[e, 1] = h2[:, 128:]
        hp2_ref[e, 0] = dinv * h2[:, :128]
        hp2_ref[e, 1] = dinv * h2[:, 128:]


def _k2(scat, hg, cnt, bc, wa, ba, wb, bb, wc2):
    bs = 1000
    return pl.pallas_call(
        _k2_body,
        grid=(N // bs,),
        in_specs=[
            pl.BlockSpec((2, 2, bs, 128), lambda i: (0, 0, i, 0)),
            pl.BlockSpec((2, 2, bs, 128), lambda i: (0, 0, i, 0)),
            pl.BlockSpec((2, bs, 2), lambda i: (0, i, 0)),
            pl.BlockSpec((2, H), lambda i: (0, 0)),
            pl.BlockSpec((2 * H, H), lambda i: (0, 0)),
            pl.BlockSpec((1, H), lambda i: (0, 0)),
            pl.BlockSpec((H, H), lambda i: (0, 0)),
            pl.BlockSpec((1, H), lambda i: (0, 0)),
            pl.BlockSpec((2, H, H), lambda i: (0, 0, 0)),
        ],
        out_specs=[
            pl.BlockSpec((2, 2, bs, 128), lambda i: (0, 0, i, 0)),
            pl.BlockSpec((2, 2, bs, 128), lambda i: (0, 0, i, 0)),
        ],
        out_shape=[
            jax.ShapeDtypeStruct((2, 2, N, 128), jnp.float32),
            jax.ShapeDtypeStruct((2, 2, N, 128), jnp.float32),
        ],
    )(scat, hg, cnt, bc, wa, ba, wb, bb, wc2)


def _k3_body(scat_ref, hg_ref, cnt_ref, bc_ref, wa_ref, ba_ref, wb_ref,
             bb_ref, i1_ref, i2_ref, g1_ref, g2_ref, c1_ref, c2_ref):
    i = pl.program_id(0)
    cat = _gcn_cat(scat_ref[...], hg_ref[...], cnt_ref, bc_ref)
    u = jnp.maximum(
        jnp.dot(cat, wa_ref[...], preferred_element_type=jnp.float32)
        + ba_ref[0][None, :], 0.0)
    h = jnp.dot(u, wb_ref[...], preferred_element_type=jnp.float32) \
        + bb_ref[0][None, :]

    @pl.when(i == 0)
    def _():
        g1_ref[...] = jnp.zeros_like(g1_ref)
        g2_ref[...] = jnp.zeros_like(g2_ref)
        c1_ref[...] = jnp.zeros_like(c1_ref)
        c2_ref[...] = jnp.zeros_like(c2_ref)

    bs = h.shape[0]
    iota = lax.broadcasted_iota(jnp.int32, (S, bs), 0)
    for idx_ref, g_ref, c_ref in ((i1_ref, g1_ref, c1_ref),
                                  (i2_ref, g2_ref, c2_ref)):
        oh = (iota == idx_ref[0, 0][None, :]).astype(jnp.float32)
        g_ref[...] += jnp.dot(oh, h, preferred_element_type=jnp.float32)
        c_ref[...] += jnp.sum(oh, axis=1)[:, None]


def _k3(scat, hg, cnt, bc, wa, ba, wb, bb, i1, i2):
    bs = 1000
    return pl.pallas_call(
        _k3_body,
        grid=(N // bs,),
        in_specs=[
            pl.BlockSpec((2, 2, bs, 128), lambda i: (0, 0, i, 0)),
            pl.BlockSpec((2, 2, bs, 128), lambda i: (0, 0, i, 0)),
            pl.BlockSpec((2, bs, 2), lambda i: (0, i, 0)),
            pl.BlockSpec((2, H), lambda i: (0, 0)),
            pl.BlockSpec((2 * H, H), lambda i: (0, 0)),
            pl.BlockSpec((1, H), lambda i: (0, 0)),
            pl.BlockSpec((H, H), lambda i: (0, 0)),
            pl.BlockSpec((1, H), lambda i: (0, 0)),
            pl.BlockSpec((1, 1, bs), lambda i: (i, 0, 0)),
            pl.BlockSpec((1, 1, bs), lambda i: (i, 0, 0)),
        ],
        out_specs=[
            pl.BlockSpec((S, H), lambda i: (0, 0)),
            pl.BlockSpec((S, H), lambda i: (0, 0)),
            pl.BlockSpec((S, 128), lambda i: (0, 0)),
            pl.BlockSpec((S, 128), lambda i: (0, 0)),
        ],
        out_shape=[
            jax.ShapeDtypeStruct((S, H), jnp.float32),
            jax.ShapeDtypeStruct((S, H), jnp.float32),
            jax.ShapeDtypeStruct((S, 128), jnp.float32),
            jax.ShapeDtypeStruct((S, 128), jnp.float32),
        ],
    )(scat, hg, cnt, bc, wa, ba, wb, bb, i1, i2)


def _k4_body(g1a, g2a, c1a, c2a, g1b, g2b, c1b, c2b,
             wma, bma, wmb, bmb, wd1, bd1, wd2, bd2, wd3, bd3, out):
    def emb(g1, g2, c1, c2):
        m1 = g1[...] / jnp.maximum(c1[:, 0:1], 1.0)
        m2 = g2[...] / jnp.maximum(c2[:, 0:1], 1.0)
        u = jnp.maximum(
            jnp.dot(m1, wma[:H], preferred_element_type=jnp.float32)
            + jnp.dot(m2, wma[H:], preferred_element_type=jnp.float32)
            + bma[0][None, :], 0.0)
        v = jnp.dot(u, wmb[...], preferred_element_type=jnp.float32) \
            + bmb[0][None, :]
        return jnp.sum(v, axis=0, keepdims=True)

    z = emb(g1a, g2a, c1a, c2a) + emb(g1b, g2b, c1b, c2b)
    z = jnp.maximum(jnp.dot(z, wd1[...], preferred_element_type=jnp.float32)
                    + bd1[0][None, :], 0.0)
    z = jnp.maximum(jnp.dot(z, wd2[...], preferred_element_type=jnp.float32)
                    + bd2[0][None, :], 0.0)
    out[...] = jnp.dot(z, wd3[...], preferred_element_type=jnp.float32) \
        + bd3[0][None, :]


def _k4(args):
    return pl.pallas_call(
        _k4_body,
        out_shape=jax.ShapeDtypeStruct((1, 1), jnp.float32),
    )(*args)


def _pad_edges(ei):
    src = ei[0]
    dst = ei[1]
    pad_s = (jnp.arange(PAD, dtype=jnp.int32) % N)
    pad_d = N + (jnp.arange(PAD, dtype=jnp.int32) % (NP - N))
    src_p = jnp.concatenate([src, pad_s])
    dst_p = jnp.concatenate([dst, pad_d])
    return src_p, dst_p


def _srcx(src_p, k):
    return jnp.concatenate([src_p + (2 * k) * N, src_p + (2 * k + 1) * N])


def kernel(x1, x2, edge1_1, edge1_2, edge2_1, edge2_2, index1_1, index1_2,
           index2_1, index2_2, Wc11, bc11, Wc12, bc12, Wc21, bc21, Wc22,
           bc22, W_m1a, b_m1a, W_m1b, b_m1b, W_m2a, b_m2a, W_m2b, b_m2b,
           W_ma, b_ma, W_mb, b_mb, Wd1, bd1, Wd2, bd2, Wd3, bd3):
    f32 = jnp.float32
    sA1, dA1 = _pad_edges(edge1_1)
    sA2, dA2 = _pad_edges(edge1_2)
    sB1, dB1 = _pad_edges(edge2_1)
    sB2, dB2 = _pad_edges(edge2_2)

    ones_h = jnp.ones((HB, 8), f32)
    z1 = jnp.zeros((ROWS_T, 8), f32)
    z2 = jnp.zeros((ROWS_T, 128), f32)

    _deg_call, _scat_call = _sc_kernels()

    def counts(s1, d1, s2, d2):
        ones_r = jnp.ones((4 * N, 128), f32)
        raw = _scat_call(ones_r, _srcx(s1, 0), d1, _srcx(s2, 1), d2,
                         z2).reshape(4, NP, 128)
        zc = jnp.zeros((NP,), f32)
        return jnp.stack([jnp.stack([raw[0, :, 0], zc], -1),
                          jnp.stack([raw[2, :, 0], zc], -1)])

    cntA = counts(sA1, dA1, sA2, dA2)
    cntB = counts(sB1, dB1, sB2, dB2)

    wc1A = jnp.stack([Wc11, Wc12])
    wc2A = jnp.stack([Wc21, Wc22])
    bcA = jnp.stack([bc11, bc12])
    bc2A = jnp.stack([bc21, bc22])

    i1A = index1_1.reshape(10, 1, 1000)
    i2A = index1_2.reshape(10, 1, 1000)
    i1B = index2_1.reshape(10, 1, 1000)
    i2B = index2_2.reshape(10, 1, 1000)

    def branch(x, s1, d1, s2, d2, cntb, i1, i2):
        hg, hp = _k1(x, wc1A, cntb)
        scat1 = _scat_call(hp.reshape(4 * N, 128), _srcx(s1, 0), d1,
                           _srcx(s2, 1), d2, z2)
        hg2, hp2 = _k2(scat1.reshape(2, 2, NP, 128), hg, cntb, bcA,
                       W_m1a, b_m1a.reshape(1, H), W_m1b,
                       b_m1b.reshape(1, H), wc2A)
        scat2 = _scat_call(hp2.reshape(4 * N, 128), _srcx(s1, 0), d1,
                           _srcx(s2, 1), d2, z2)
        return _k3(scat2.reshape(2, 2, NP, 128), hg2, cntb, bc2A,
                   W_m2a, b_m2a.reshape(1, H), W_m2b,
                   b_m2b.reshape(1, H), i1, i2)

    g1a, g2a, c1a, c2a = branch(x1, sA1, dA1, sA2, dA2, cntA, i1A, i2A)
    g1b, g2b, c1b, c2b = branch(x2, sB1, dB1, sB2, dB2, cntB, i1B, i2B)

    out = _k4((g1a, g2a, c1a, c2a, g1b, g2b, c1b, c2b,
               W_ma, b_ma.reshape(1, H), W_mb, b_mb.reshape(1, 7),
               Wd1, bd1.reshape(1, 16), Wd2, bd2.reshape(1, 16),
               Wd3, bd3.reshape(1, 1)))
    return out.reshape((1,))
